# baseline (speedup 1.0000x reference)
import functools

import jax
import jax.numpy as jnp
from jax import lax
from jax.experimental import pallas as pl
from jax.experimental.pallas import tpu as pltpu
from jax.experimental.pallas import tpu_sc as plsc

_CHUNK = 128
_ROW = 128
_L = 16


def _gather_kernel(B, D, NC, NW):
    b_per_w = B // NW
    n_chunks = b_per_w // _CHUNK
    per_row = _ROW // D
    shift = per_row.bit_length() - 1
    mesh = plsc.VectorSubcoreMesh(core_axis_name="c", subcore_axis_name="s")

    @functools.partial(
        pl.kernel,
        mesh=mesh,
        out_type=jax.ShapeDtypeStruct((B, D), jnp.float32),
        compiler_params=pltpu.CompilerParams(needs_layout_passes=False),
        scratch_types=[
            pltpu.VMEM((b_per_w,), jnp.int32),
            pltpu.VMEM((n_chunks, _CHUNK), jnp.int32),
            pltpu.VMEM((2, _CHUNK, _ROW), jnp.float32),
            pltpu.VMEM((b_per_w, D), jnp.float32),
            pltpu.SemaphoreType.DMA,
            pltpu.SemaphoreType.DMA,
        ],
    )
    def k(idx_hbm, table_hbm, out_hbm, idx_v, big_v, rows_v, out_v, sem0, sem1):
        wid = lax.axis_index("s") * NC + lax.axis_index("c")
        sems = (sem0, sem1)
        pltpu.sync_copy(idx_hbm.at[wid], idx_v)
        for j in range(n_chunks):
            for t in range(_CHUNK // _L):
                v = idx_v[pl.ds(j * _CHUNK + t * _L, _L)]
                big_v[j, pl.ds(t * _L, _L)] = lax.shift_right_logical(v, shift)

        def start_gather(j):
            return pltpu.async_copy(
                table_hbm.at[big_v.at[j]], rows_v.at[j % 2], sems[j % 2]
            )

        lanes = lax.iota(jnp.int32, _L)

        def extract_chunk(j):
            buf = rows_v.at[j % 2]

            def block(t, _):
                j0 = j * _CHUNK + t * _L
                local16 = t * _L + lanes
                glob16 = j0 + lanes
                jv = idx_v[pl.ds(j0, _L)]
                col0 = (jv & (per_row - 1)) * D
                for c in range(D):
                    vals = plsc.load_gather(buf, [local16, col0 + c])
                    plsc.store_scatter(
                        out_v, [glob16, jnp.full((_L,), c, jnp.int32)], vals
                    )
                return _

            lax.fori_loop(0, _CHUNK // _L, block, 0)

        inflight = start_gather(0)
        for j in range(n_chunks):
            inflight.wait()
            if j + 1 < n_chunks:
                inflight = start_gather(j + 1)
            extract_chunk(j)
        pltpu.sync_copy(out_v, out_hbm.at[pl.ds(wid * b_per_w, b_per_w)])

    return k


def kernel(domains, table):
    B, = domains.shape
    V, D = table.shape
    info = plsc.get_sparse_core_info()
    NC, NS = info.num_cores, info.num_subcores
    NW = NC * NS
    table_big = table.reshape(V * D // _ROW, _ROW)
    idx2 = domains.reshape(NW, B // NW)
    k = _gather_kernel(B, D, NC, NW)
    return k(idx2, table_big)


# speedup vs baseline: 1.1955x; 1.1955x over previous
import functools

import jax
import jax.numpy as jnp
from jax import lax
from jax.experimental import pallas as pl
from jax.experimental.pallas import tpu as pltpu
from jax.experimental.pallas import tpu_sc as plsc

_CHUNK = 128


def _gather_kernel(B, D, NC, NW):
    b_per_w = B // NW
    n_chunks = b_per_w // _CHUNK
    mesh = plsc.VectorSubcoreMesh(core_axis_name="c", subcore_axis_name="s")

    @functools.partial(
        pl.kernel,
        mesh=mesh,
        out_type=jax.ShapeDtypeStruct((B, D), jnp.float32),
        compiler_params=pltpu.CompilerParams(use_tc_tiling_on_sc=False),
        scratch_types=[
            pltpu.VMEM((n_chunks, _CHUNK), jnp.int32),
            pltpu.VMEM((b_per_w, D), jnp.float32),
            pltpu.SemaphoreType.DMA,
        ],
    )
    def k(idx_hbm, table_hbm, out_hbm, idx_v, rows_v, sem):
        wid = lax.axis_index("s") * NC + lax.axis_index("c")
        pltpu.sync_copy(idx_hbm.at[wid], idx_v)
        copies = [
            pltpu.async_copy(
                table_hbm.at[idx_v.at[j]],
                rows_v.at[pl.ds(j * _CHUNK, _CHUNK)],
                sem,
            )
            for j in range(n_chunks)
        ]
        for c in copies:
            c.wait()
        pltpu.sync_copy(rows_v, out_hbm.at[pl.ds(wid * b_per_w, b_per_w)])

    return k


def kernel(domains, table):
    B, = domains.shape
    V, D = table.shape
    info = plsc.get_sparse_core_info()
    NC, NS = info.num_cores, info.num_subcores
    NW = NC * NS
    b_per_w = B // NW
    idx3 = domains.reshape(NW, b_per_w // _CHUNK, _CHUNK)
    k = _gather_kernel(B, D, NC, NW)
    return k(idx3, table)
